# baseline (speedup 1.0000x reference)
import functools

import jax
import jax.numpy as jnp
from jax import lax
from jax.experimental import pallas as pl
from jax.experimental.pallas import tpu as pltpu
from jax.experimental.pallas import tpu_sc as plsc

N = 10000
E = 640000
F_IN = 128
C = 40
EPS = 1e-5

NC = 2
NS = 16
NW = NC * NS
CH = 128
CHUNKS_PER_TILE = 160
CHUNKS = NW * CHUNKS_PER_TILE
E_PAD = CHUNKS * CH
PHASES = 2
CPP = CHUNKS_PER_TILE // PHASES
ROWS_PER_TILE = 632
N_ACC = NS * ROWS_PER_TILE
DEG_F = 16

_mesh = plsc.VectorSubcoreMesh(core_axis_name="c", subcore_axis_name="s",
                               num_cores=NC, num_subcores=NS)


def _zero_acc_slice(rows_v, acc_sh, s, f):
    def zrow(i, _):
        for j in range(f // 16):
            rows_v[i, pl.ds(j * 16, 16)] = jnp.zeros((16,), jnp.float32)
        return 0

    lax.fori_loop(0, CH, zrow, 0)
    base = s * ROWS_PER_TILE
    for off in (0, 128, 256, 384, ROWS_PER_TILE - 128):
        pltpu.sync_copy(rows_v, acc_sh.at[pl.ds(base + off, 128)])


def _make_agg(f):
    def body(tbl_hbm, src_hbm, dst_hbm, out_hbm, tbl_sh, acc_sh, src_v,
             dst_v, rows_v, gsem0, gsem1):
        c = lax.axis_index("c")
        s = lax.axis_index("s")
        wid = s * NC + c

        _zero_acc_slice(rows_v.at[0], acc_sh, s, f)

        base = s * ROWS_PER_TILE
        pltpu.sync_copy(tbl_hbm.at[pl.ds(base, ROWS_PER_TILE)],
                        tbl_sh.at[pl.ds(base, ROWS_PER_TILE)])

        plsc.subcore_barrier()

        for phase in range(PHASES):
            pbase = wid * CHUNKS_PER_TILE + phase * CPP
            pltpu.sync_copy(src_hbm.at[pl.ds(pbase, CPP)], src_v)
            pltpu.sync_copy(dst_hbm.at[pl.ds(pbase, CPP)], dst_v)

            pltpu.async_copy(tbl_sh.at[src_v.at[0]], rows_v.at[0], gsem0)

            @pl.loop(0, CPP, step=2)
            def pair(j0):
                pltpu.async_copy(tbl_sh.at[src_v.at[j0 + 1]], rows_v.at[1],
                                 gsem1)
                pltpu.make_async_copy(tbl_sh.at[src_v.at[j0]], rows_v.at[0],
                                      gsem0).wait()
                pltpu.sync_copy(rows_v.at[0], acc_sh.at[dst_v.at[j0]],
                                add=True)

                @pl.when(j0 + 2 < CPP)
                def _():
                    pltpu.async_copy(tbl_sh.at[src_v.at[j0 + 2]],
                                     rows_v.at[0], gsem0)

                pltpu.make_async_copy(tbl_sh.at[src_v.at[j0 + 1]],
                                      rows_v.at[1], gsem1).wait()
                pltpu.sync_copy(rows_v.at[1], acc_sh.at[dst_v.at[j0 + 1]],
                                add=True)

        plsc.subcore_barrier()

        pltpu.sync_copy(acc_sh.at[pl.ds(base, ROWS_PER_TILE)],
                        out_hbm.at[c, pl.ds(base, ROWS_PER_TILE)])

    return pl.kernel(
        body,
        out_type=jax.ShapeDtypeStruct((NC, N_ACC, f), jnp.float32),
        mesh=_mesh,
        compiler_params=pltpu.CompilerParams(use_tc_tiling_on_sc=False),
        scratch_types=[
            pltpu.VMEM_SHARED((N_ACC, f), jnp.float32),
            pltpu.VMEM_SHARED((N_ACC, f), jnp.float32),
            pltpu.VMEM((CPP, CH), jnp.int32),
            pltpu.VMEM((CPP, CH), jnp.int32),
            pltpu.VMEM((2, CH, f), jnp.float32),
            pltpu.SemaphoreType.DMA,
            pltpu.SemaphoreType.DMA,
        ],
    )


def _make_deg():
    def body(dst_hbm, out_hbm, acc_sh, dst_v, rows_v):
        c = lax.axis_index("c")
        s = lax.axis_index("s")
        wid = s * NC + c

        _zero_acc_slice(rows_v, acc_sh, s, DEG_F)

        def onerow(i, _):
            rows_v[i, pl.ds(0, 16)] = jnp.ones((16,), jnp.float32)
            return 0

        lax.fori_loop(0, CH, onerow, 0)

        pltpu.sync_copy(dst_hbm.at[pl.ds(wid * CHUNKS_PER_TILE,
                                         CHUNKS_PER_TILE)], dst_v)
        plsc.subcore_barrier()

        def step(j, _):
            pltpu.sync_copy(rows_v, acc_sh.at[dst_v.at[j]], add=True)
            return 0

        lax.fori_loop(0, CHUNKS_PER_TILE, step, 0)
        plsc.subcore_barrier()

        base = s * ROWS_PER_TILE
        pltpu.sync_copy(acc_sh.at[pl.ds(base, ROWS_PER_TILE)],
                        out_hbm.at[c, pl.ds(base, ROWS_PER_TILE)])

    return pl.kernel(
        body,
        out_type=jax.ShapeDtypeStruct((NC, N_ACC, DEG_F), jnp.float32),
        mesh=_mesh,
        compiler_params=pltpu.CompilerParams(use_tc_tiling_on_sc=False),
        scratch_types=[
            pltpu.VMEM_SHARED((N_ACC, DEG_F), jnp.float32),
            pltpu.VMEM((CHUNKS_PER_TILE, CH), jnp.int32),
            pltpu.VMEM((CH, DEG_F), jnp.float32),
        ],
    )


_agg64 = _make_agg(64)
_agg32 = _make_agg(32)
_deg = _make_deg()


def _tc(fn, *out_shapes):
    return pl.pallas_call(
        fn, out_shape=[jax.ShapeDtypeStruct(s, jnp.float32)
                       for s in out_shapes])


def _bn_relu(x, g, b):
    m = jnp.mean(x, axis=0)
    v = jnp.mean((x - m) ** 2, axis=0)
    return jnp.maximum((x - m) * lax.rsqrt(v + EPS) * g + b, 0.0)


def _dot(a, b):
    return jnp.dot(a, b, preferred_element_type=jnp.float32)


def _pad_rows(h):
    return jnp.concatenate(
        [h, jnp.zeros((N_ACC - N, h.shape[1]), h.dtype)], axis=0)


def _t0a_body(x, w1, h1_o):
    h1_o[...] = _dot(x[...], w1[...])


def _t0_body(degp, h1, dinv_o, hp1_o):
    deg = degp[0, :N, 0:1] + degp[1, :N, 0:1] + 1.0
    dinv = lax.rsqrt(deg)
    dinv_o[...] = dinv
    hp1_o[...] = _pad_rows(h1[...] * dinv)


def _t1_body(aggp, hp1, dinv, b1, g1, be1, hp2_o):
    s = aggp[0, :N, :] + aggp[1, :N, :] + hp1[:N, :]
    c1 = dinv[...] * s + b1[...]
    hp2_o[...] = _pad_rows(dinv[...] * _bn_relu(c1, g1[...], be1[...]))


def _t2_body(aggp, hp2, dinv, w2, b2, g2, be2, w3, hp3_o):
    x2 = dinv[...] * (aggp[0, :N, :] + aggp[1, :N, :] + hp2[:N, :])
    c2 = _dot(x2, w2[...]) + b2[...]
    h3 = _bn_relu(c2, g2[...], be2[...])
    hp3_o[...] = _pad_rows(dinv[...] * _dot(h3, w3[...]))


def _t3_body(aggp, hp3, dinv, b3, g3, be3, w4, hp4_o):
    c3 = dinv[...] * (aggp[0, :N, :] + aggp[1, :N, :] + hp3[:N, :]) + b3[...]
    h4 = _bn_relu(c3, g3[...], be3[...])
    hp4_o[...] = _pad_rows(dinv[...] * _dot(h4, w4[...]))


def _t4_body(aggp, hp4, dinv, b4, g4, be4, hp5_o):
    c4 = dinv[...] * (aggp[0, :N, :] + aggp[1, :N, :] + hp4[:N, :]) + b4[...]
    hp5_o[...] = _pad_rows(dinv[...] * _bn_relu(c4, g4[...], be4[...]))


def _t4b_body(hp5, dinv, w5, b5, term2_o):
    term2_o[...] = _dot(dinv[...] * hp5[:N, :], w5[...]) + b5[...]


def _t5_body(aggp, dinv, w5, term2, out_o):
    x5 = dinv[...] * (aggp[0, :N, :] + aggp[1, :N, :])
    out_o[...] = _dot(x5, w5[...]) + term2[...]


def kernel(x, edge_index, W1, b1, W2, b2, W3, b3, W4, b4, W5, b5,
           g1, be1, g2, be2, g3, be3, g4, be4):
    src = edge_index[0]
    dst = edge_index[1]
    pad = E_PAD - E
    srcI = jnp.concatenate(
        [src, jnp.zeros((pad,), jnp.int32)]).reshape(CHUNKS, CH)
    dstI = jnp.concatenate(
        [dst, jnp.full((pad,), N, jnp.int32)]).reshape(CHUNKS, CH)

    degp = _deg(dstI)
    h1, = _tc(_t0a_body, (N, 64))(x, W1)
    dinv, hp1 = _tc(_t0_body, (N, 1), (N_ACC, 64))(degp, h1)
    a1 = _agg64(hp1, srcI, dstI)
    hp2, = _tc(_t1_body, (N_ACC, 64))(a1, hp1, dinv, b1, g1, be1)
    a2 = _agg64(hp2, srcI, dstI)
    hp3, = _tc(_t2_body, (N_ACC, 64))(a2, hp2, dinv, W2, b2, g2, be2, W3)
    a3 = _agg64(hp3, srcI, dstI)
    hp4, = _tc(_t3_body, (N_ACC, 32))(a3, hp3, dinv, b3, g3, be3, W4)
    a4 = _agg32(hp4, srcI, dstI)
    hp5, = _tc(_t4_body, (N_ACC, 32))(a4, hp4, dinv, b4, g4, be4)
    a5 = _agg32(hp5, srcI, dstI)
    term2, = _tc(_t4b_body, (N, C))(hp5, dinv, W5, b5)
    out, = _tc(_t5_body, (N, C))(a5, dinv, W5, term2)
    return out


# speedup vs baseline: 1.1063x; 1.1063x over previous
import functools

import jax
import jax.numpy as jnp
from jax import lax
from jax.experimental import pallas as pl
from jax.experimental.pallas import tpu as pltpu
from jax.experimental.pallas import tpu_sc as plsc

N = 10000
E = 640000
F_IN = 128
C = 40
EPS = 1e-5

NC = 2
NS = 16
NW = NC * NS
CH = 128
CHUNKS_PER_TILE = 160
CHUNKS = NW * CHUNKS_PER_TILE
E_PAD = CHUNKS * CH
PHASES = 4
CPP = CHUNKS_PER_TILE // PHASES
NBUF = 4
ROWS_PER_TILE = 632
N_ACC = NS * ROWS_PER_TILE
DEG_F = 16

_mesh = plsc.VectorSubcoreMesh(core_axis_name="c", subcore_axis_name="s",
                               num_cores=NC, num_subcores=NS)


def _zero_acc_slice(rows_v, acc_sh, s, f):
    def zrow(i, _):
        for j in range(f // 16):
            rows_v[i, pl.ds(j * 16, 16)] = jnp.zeros((16,), jnp.float32)
        return 0

    lax.fori_loop(0, CH, zrow, 0)
    base = s * ROWS_PER_TILE
    for off in (0, 128, 256, 384, ROWS_PER_TILE - 128):
        pltpu.sync_copy(rows_v, acc_sh.at[pl.ds(base + off, 128)])


def _make_agg(f):
    def body(tbl_hbm, src_hbm, dst_hbm, out_hbm, tbl_sh, acc_sh, src_v,
             dst_v, rows_v, g0, g1, g2, g3, s0, s1, s2, s3):
        gsem = (g0, g1, g2, g3)
        ssem = (s0, s1, s2, s3)
        c = lax.axis_index("c")
        s = lax.axis_index("s")
        wid = s * NC + c

        _zero_acc_slice(rows_v.at[0], acc_sh, s, f)

        base = s * ROWS_PER_TILE
        pltpu.sync_copy(tbl_hbm.at[pl.ds(base, ROWS_PER_TILE)],
                        tbl_sh.at[pl.ds(base, ROWS_PER_TILE)])

        plsc.subcore_barrier()

        for phase in range(PHASES):
            pbase = wid * CHUNKS_PER_TILE + phase * CPP
            pltpu.sync_copy(src_hbm.at[pl.ds(pbase, CPP)], src_v)
            pltpu.sync_copy(dst_hbm.at[pl.ds(pbase, CPP)], dst_v)

            pltpu.async_copy(tbl_sh.at[src_v.at[0]], rows_v.at[0], gsem[0])
            pltpu.async_copy(tbl_sh.at[src_v.at[1]], rows_v.at[1], gsem[1])

            @pl.loop(0, CPP, step=NBUF)
            def quad(j0):
                for u in range(NBUF):
                    j = j0 + u
                    b = u % NBUF
                    bp = (u + 2) % NBUF
                    pltpu.make_async_copy(tbl_sh.at[src_v.at[j]],
                                          rows_v.at[b], gsem[b]).wait()
                    pltpu.async_copy(rows_v.at[b], acc_sh.at[dst_v.at[j]],
                                     ssem[b], add=True)

                    @pl.when(j >= 2)
                    def _():
                        pltpu.make_async_copy(
                            rows_v.at[bp], acc_sh.at[dst_v.at[j]],
                            ssem[bp]).wait()

                    @pl.when(j + 2 < CPP)
                    def _():
                        pltpu.async_copy(tbl_sh.at[src_v.at[j + 2]],
                                         rows_v.at[bp], gsem[bp])

            for u in range(2):
                b = (CPP - 2 + u) % NBUF
                pltpu.make_async_copy(rows_v.at[b],
                                      acc_sh.at[dst_v.at[CPP - 2 + u]],
                                      ssem[b]).wait()

        plsc.subcore_barrier()

        pltpu.sync_copy(acc_sh.at[pl.ds(base, ROWS_PER_TILE)],
                        out_hbm.at[c, pl.ds(base, ROWS_PER_TILE)])

    return pl.kernel(
        body,
        out_type=jax.ShapeDtypeStruct((NC, N_ACC, f), jnp.float32),
        mesh=_mesh,
        compiler_params=pltpu.CompilerParams(use_tc_tiling_on_sc=False),
        scratch_types=[
            pltpu.VMEM_SHARED((N_ACC, f), jnp.float32),
            pltpu.VMEM_SHARED((N_ACC, f), jnp.float32),
            pltpu.VMEM((CPP, CH), jnp.int32),
            pltpu.VMEM((CPP, CH), jnp.int32),
            pltpu.VMEM((NBUF, CH, f), jnp.float32),
        ] + [pltpu.SemaphoreType.DMA] * (2 * NBUF),
    )


def _make_deg():
    def body(dst_hbm, out_hbm, acc_sh, dst_v, rows_v):
        c = lax.axis_index("c")
        s = lax.axis_index("s")
        wid = s * NC + c

        _zero_acc_slice(rows_v, acc_sh, s, DEG_F)

        def onerow(i, _):
            rows_v[i, pl.ds(0, 16)] = jnp.ones((16,), jnp.float32)
            return 0

        lax.fori_loop(0, CH, onerow, 0)

        pltpu.sync_copy(dst_hbm.at[pl.ds(wid * CHUNKS_PER_TILE,
                                         CHUNKS_PER_TILE)], dst_v)
        plsc.subcore_barrier()

        def step(j, _):
            pltpu.sync_copy(rows_v, acc_sh.at[dst_v.at[j]], add=True)
            return 0

        lax.fori_loop(0, CHUNKS_PER_TILE, step, 0)
        plsc.subcore_barrier()

        base = s * ROWS_PER_TILE
        pltpu.sync_copy(acc_sh.at[pl.ds(base, ROWS_PER_TILE)],
                        out_hbm.at[c, pl.ds(base, ROWS_PER_TILE)])

    return pl.kernel(
        body,
        out_type=jax.ShapeDtypeStruct((NC, N_ACC, DEG_F), jnp.float32),
        mesh=_mesh,
        compiler_params=pltpu.CompilerParams(use_tc_tiling_on_sc=False),
        scratch_types=[
            pltpu.VMEM_SHARED((N_ACC, DEG_F), jnp.float32),
            pltpu.VMEM((CHUNKS_PER_TILE, CH), jnp.int32),
            pltpu.VMEM((CH, DEG_F), jnp.float32),
        ],
    )


_agg64 = _make_agg(64)
_agg32 = _make_agg(32)
_deg = _make_deg()


def _tc(fn, *out_shapes):
    return pl.pallas_call(
        fn, out_shape=[jax.ShapeDtypeStruct(s, jnp.float32)
                       for s in out_shapes])


def _bn_relu(x, g, b):
    m = jnp.mean(x, axis=0)
    v = jnp.mean((x - m) ** 2, axis=0)
    return jnp.maximum((x - m) * lax.rsqrt(v + EPS) * g + b, 0.0)


def _dot(a, b):
    return jnp.dot(a, b, preferred_element_type=jnp.float32)


def _pad_rows(h):
    return jnp.concatenate(
        [h, jnp.zeros((N_ACC - N, h.shape[1]), h.dtype)], axis=0)


def _t0a_body(x, w1, h1_o):
    h1_o[...] = _dot(x[...], w1[...])


def _t0_body(degp, h1, dinv_o, hp1_o):
    deg = degp[0, :N, 0:1] + degp[1, :N, 0:1] + 1.0
    dinv = lax.rsqrt(deg)
    dinv_o[...] = dinv
    hp1_o[...] = _pad_rows(h1[...] * dinv)


def _t1_body(aggp, hp1, dinv, b1, g1, be1, hp2_o):
    s = aggp[0, :N, :] + aggp[1, :N, :] + hp1[:N, :]
    c1 = dinv[...] * s + b1[...]
    hp2_o[...] = _pad_rows(dinv[...] * _bn_relu(c1, g1[...], be1[...]))


def _t2_body(aggp, hp2, dinv, w2, b2, g2, be2, w3, hp3_o):
    x2 = dinv[...] * (aggp[0, :N, :] + aggp[1, :N, :] + hp2[:N, :])
    c2 = _dot(x2, w2[...]) + b2[...]
    h3 = _bn_relu(c2, g2[...], be2[...])
    hp3_o[...] = _pad_rows(dinv[...] * _dot(h3, w3[...]))


def _t3_body(aggp, hp3, dinv, b3, g3, be3, w4, hp4_o):
    c3 = dinv[...] * (aggp[0, :N, :] + aggp[1, :N, :] + hp3[:N, :]) + b3[...]
    h4 = _bn_relu(c3, g3[...], be3[...])
    hp4_o[...] = _pad_rows(dinv[...] * _dot(h4, w4[...]))


def _t4_body(aggp, hp4, dinv, b4, g4, be4, hp5_o):
    c4 = dinv[...] * (aggp[0, :N, :] + aggp[1, :N, :] + hp4[:N, :]) + b4[...]
    hp5_o[...] = _pad_rows(dinv[...] * _bn_relu(c4, g4[...], be4[...]))


def _t4b_body(hp5, dinv, w5, b5, term2_o):
    term2_o[...] = _dot(dinv[...] * hp5[:N, :], w5[...]) + b5[...]


def _t5_body(aggp, dinv, w5, term2, out_o):
    x5 = dinv[...] * (aggp[0, :N, :] + aggp[1, :N, :])
    out_o[...] = _dot(x5, w5[...]) + term2[...]


def kernel(x, edge_index, W1, b1, W2, b2, W3, b3, W4, b4, W5, b5,
           g1, be1, g2, be2, g3, be3, g4, be4):
    src = edge_index[0]
    dst = edge_index[1]
    pad = E_PAD - E
    srcI = jnp.concatenate(
        [src, jnp.zeros((pad,), jnp.int32)]).reshape(CHUNKS, CH)
    dstI = jnp.concatenate(
        [dst, jnp.full((pad,), N, jnp.int32)]).reshape(CHUNKS, CH)

    degp = _deg(dstI)
    h1, = _tc(_t0a_body, (N, 64))(x, W1)
    dinv, hp1 = _tc(_t0_body, (N, 1), (N_ACC, 64))(degp, h1)
    a1 = _agg64(hp1, srcI, dstI)
    hp2, = _tc(_t1_body, (N_ACC, 64))(a1, hp1, dinv, b1, g1, be1)
    a2 = _agg64(hp2, srcI, dstI)
    hp3, = _tc(_t2_body, (N_ACC, 64))(a2, hp2, dinv, W2, b2, g2, be2, W3)
    a3 = _agg64(hp3, srcI, dstI)
    hp4, = _tc(_t3_body, (N_ACC, 32))(a3, hp3, dinv, b3, g3, be3, W4)
    a4 = _agg32(hp4, srcI, dstI)
    hp5, = _tc(_t4_body, (N_ACC, 32))(a4, hp4, dinv, b4, g4, be4)
    a5 = _agg32(hp5, srcI, dstI)
    term2, = _tc(_t4b_body, (N, C))(hp5, dinv, W5, b5)
    out, = _tc(_t5_body, (N, C))(a5, dinv, W5, term2)
    return out
